# Initial kernel scaffold; baseline (speedup 1.0000x reference)
import jax
import jax.numpy as jnp
from jax.experimental import pallas as pl


def kernel(boxes, scores):
    raise NotImplementedError("write your pallas kernel here")


# speedup vs baseline: 92.7603x; 92.7603x over previous
import jax
import jax.numpy as jnp
from jax.experimental import pallas as pl
from jax.experimental.pallas import tpu as pltpu

_N = 20000
_TOPK = 5000
_KEEP = 750
_CONF = 0.8
_T = 0.3
_B = 512
_NB = 10
_NP = _B * _NB


def _nms_body(xr, yr, Xr, Yr, sr, bc, dest_ref, ksc_ref, keep_ref):
    i = pl.program_id(0)
    f32 = jnp.float32

    @pl.when(i == 0)
    def _init():
        keep_ref[...] = (sr[...] > 0.0).astype(f32)

    base = i * _B
    px1 = bc[pl.ds(base, _B), 0:1]
    py1 = bc[pl.ds(base, _B), 1:2]
    px2 = bc[pl.ds(base, _B), 2:3]
    py2 = bc[pl.ds(base, _B), 3:4]
    pa = jnp.maximum(px2 - px1, 0.0) * jnp.maximum(py2 - py1, 0.0)

    def strip_sup(j):
        tx1 = xr[pl.ds(j, 1), :]
        ty1 = yr[pl.ds(j, 1), :]
        tx2 = Xr[pl.ds(j, 1), :]
        ty2 = Yr[pl.ds(j, 1), :]
        ta = jnp.maximum(tx2 - tx1, 0.0) * jnp.maximum(ty2 - ty1, 0.0)
        ix1 = jnp.maximum(px1, tx1)
        iy1 = jnp.maximum(py1, ty1)
        ix2 = jnp.minimum(px2, tx2)
        iy2 = jnp.minimum(py2, ty2)
        iw = jnp.maximum(ix2 - ix1, 0.0)
        ih = jnp.maximum(iy2 - iy1, 0.0)
        inter = iw * ih
        union = (pa + ta) - inter
        iou = inter / jnp.maximum(union, 1e-9)
        return (iou > _T).astype(f32)

    sup_ii = strip_sup(i)
    ci = jax.lax.broadcasted_iota(jnp.int32, (_B, _B), 1)
    ri = jax.lax.broadcasted_iota(jnp.int32, (_B, _B), 0)
    sup_ii = jnp.where(ci > ri, sup_ii, 0.0)

    k0 = keep_ref[pl.ds(i, 1), :]

    def cond(st):
        return st[1]

    def body(st):
        k, _ = st
        s = jax.lax.dot_general(k, sup_ii, (((1,), (0,)), ((), ())),
                                preferred_element_type=f32)
        kn = jnp.where(s > 0.0, 0.0, k0)
        return kn, jnp.any(kn != k)

    kfin, _ = jax.lax.while_loop(cond, body, (k0, jnp.bool_(True)))
    keep_ref[pl.ds(i, 1), :] = kfin

    def cross(j, carry):
        sup = strip_sup(j)
        s = jax.lax.dot_general(kfin, sup, (((1,), (0,)), ((), ())),
                                preferred_element_type=f32)
        kj = keep_ref[pl.ds(j, 1), :]
        keep_ref[pl.ds(j, 1), :] = jnp.where(s > 0.0, 0.0, kj)
        return carry

    jax.lax.fori_loop(i + 1, _NB, cross, 0)

    @pl.when(i == _NB - 1)
    def _fin():
        keep = keep_ref[...]
        r2 = jax.lax.broadcasted_iota(jnp.int32, (_NB, _B), 0)
        c2 = jax.lax.broadcasted_iota(jnp.int32, (_NB, _B), 1)
        pidx = r2 * _B + c2
        real = jnp.where(pidx < _TOPK, 1.0, 0.0)
        nonk = (1.0 - keep) * real

        r1 = jax.lax.broadcasted_iota(jnp.int32, (_NB, 1), 0)

        def cumsum_linear(m):
            x = m
            sh = 1
            while sh < _B:
                x = x + jnp.where(c2 >= sh, jnp.roll(x, sh, axis=1), 0.0)
                sh *= 2
            tot = x[:, _B - 1:_B]
            off = tot
            sh = 1
            while sh < _NB:
                off = off + jnp.where(r1 >= sh, jnp.roll(off, sh, axis=0), 0.0)
                sh *= 2
            return x + (off - tot)

        ck = cumsum_linear(keep)
        cn = cumsum_linear(nonk)
        nk = ck[_NB - 1:_NB, _B - 1:_B]
        dest = jnp.where(keep > 0.0, ck - 1.0, (cn - 1.0) + nk)
        ok = ((keep + nonk) > 0.0) & (dest < float(_KEEP))
        dest_ref[...] = jnp.where(ok, dest, float(_KEEP)).astype(jnp.int32)
        ksc_ref[...] = jnp.where(keep > 0.0, sr[...], -1.0)


def _nms_call(xr, yr, Xr, Yr, sr, bc):
    full = lambda i: (0, 0)
    return pl.pallas_call(
        _nms_body,
        grid=(_NB,),
        in_specs=[
            pl.BlockSpec((_NB, _B), full),
            pl.BlockSpec((_NB, _B), full),
            pl.BlockSpec((_NB, _B), full),
            pl.BlockSpec((_NB, _B), full),
            pl.BlockSpec((_NB, _B), full),
            pl.BlockSpec((_NP, 8), full),
        ],
        out_specs=(pl.BlockSpec((_NB, _B), full),
                   pl.BlockSpec((_NB, _B), full)),
        out_shape=(jax.ShapeDtypeStruct((_NB, _B), jnp.int32),
                   jax.ShapeDtypeStruct((_NB, _B), jnp.float32)),
        scratch_shapes=[pltpu.VMEM((_NB, _B), jnp.float32)],
    )(xr, yr, Xr, Yr, sr, bc)


@jax.jit
def kernel(boxes, scores):
    masked = jnp.where(scores >= _CONF, scores, -1.0)
    ts, ti = jax.lax.top_k(masked, _TOPK)
    b = jnp.take(boxes, ti, axis=0)
    bp = jnp.concatenate([b, jnp.zeros((_NP - _TOPK, 4), jnp.float32)], 0)
    sp = jnp.concatenate([ts, jnp.full((_NP - _TOPK,), -1.0, jnp.float32)], 0)
    xr = bp[:, 0].reshape(_NB, _B)
    yr = bp[:, 1].reshape(_NB, _B)
    Xr = bp[:, 2].reshape(_NB, _B)
    Yr = bp[:, 3].reshape(_NB, _B)
    sr = sp.reshape(_NB, _B)
    bc = jnp.concatenate([bp, sp[:, None], jnp.zeros((_NP, 3), jnp.float32)], 1)
    dest, ksc = _nms_call(xr, yr, Xr, Yr, sr, bc)
    data = jnp.concatenate([bp, ksc.reshape(_NP)[:, None]], 1)
    out = jnp.zeros((_KEEP + 1, 5), jnp.float32).at[dest.reshape(_NP)].set(
        data, mode="drop")
    return out[:_KEEP]
